# baseline (speedup 1.0000x reference)
import functools

import jax
import jax.numpy as jnp
from jax import lax
from jax.experimental import pallas as pl
from jax.experimental.pallas import tpu as pltpu
from jax.experimental.pallas import tpu_sc as plsc

N = 10000
C = 128
E = 320000
NC = 2
NS = 16
NW = NC * NS
NP = 10240
RPT = NP // NS
EF = E + N
EP = 339456
EPW = EP // NW
B = 104
NBLK = EPW // B
SB = NBLK // 3
VL = 16


def _sc_mesh():
    return plsc.VectorSubcoreMesh(
        core_axis_name="c", subcore_axis_name="s", num_cores=NC, num_subcores=NS
    )


_SC_PARAMS = pltpu.CompilerParams(needs_layout_passes=False)


def _sc_deg(dst3, ew3, z1d):
    @functools.partial(
        pl.kernel,
        out_type=(
            jax.ShapeDtypeStruct((NP,), jnp.float32),
            jax.ShapeDtypeStruct((NP,), jnp.float32),
        ),
        mesh=_sc_mesh(),
        compiler_params=_SC_PARAMS,
        scratch_types=[
            pltpu.VMEM((NBLK, B), jnp.int32),
            pltpu.VMEM((NBLK, B), jnp.float32),
            pltpu.VMEM_SHARED((NP,), jnp.float32),
            pltpu.SemaphoreType.DMA,
        ],
    )
    def k(dst_hbm, ew_hbm, z_hbm, out0, out1, didx_v, ew_v, deg_sh, ssem):
        cid = lax.axis_index("c")
        sid = lax.axis_index("s")
        wid = sid * NC + cid
        r0 = sid * RPT
        pltpu.sync_copy(dst_hbm.at[wid], didx_v)
        pltpu.sync_copy(ew_hbm.at[wid], ew_v)
        pltpu.sync_copy(z_hbm.at[pl.ds(r0, RPT)], deg_sh.at[pl.ds(r0, RPT)])
        plsc.subcore_barrier()

        def body(b, carry):
            pltpu.async_copy(
                ew_v.at[b], deg_sh.at[didx_v.at[b]], ssem, add=True
            )
            @pl.when(b >= 8)
            def _():
                pltpu.make_async_copy(
                    z_hbm.at[pl.ds(0, B)], ew_v.at[0], ssem
                ).wait()
            return carry

        lax.fori_loop(0, NBLK, body, 0)
        for _ in range(8):
            pltpu.make_async_copy(
                z_hbm.at[pl.ds(0, B)], ew_v.at[0], ssem
            ).wait()
        plsc.subcore_barrier()

        @pl.when(cid == 0)
        def _():
            pltpu.sync_copy(deg_sh.at[pl.ds(r0, RPT)], out0.at[pl.ds(r0, RPT)])

        @pl.when(cid == 1)
        def _():
            pltpu.sync_copy(deg_sh.at[pl.ds(r0, RPT)], out1.at[pl.ds(r0, RPT)])

    return k(dst3, ew3, z1d)


def _tc_prep(x, conv_w, deg0, deg1):
    RB = 1024
    grid = NP // RB

    def body(x_ref, w_ref, d0_ref, d1_ref, xws_ref, dis_ref):
        xw = jnp.dot(
            x_ref[...], w_ref[...], preferred_element_type=jnp.float32
        )
        deg = d0_ref[...] + d1_ref[...]
        dis = lax.rsqrt(jnp.maximum(deg, 1e-12))
        dis_ref[...] = dis
        xws_ref[...] = xw * dis[:, None]

    return pl.pallas_call(
        body,
        grid=(grid,),
        in_specs=[
            pl.BlockSpec((RB, C), lambda i: (i, 0)),
            pl.BlockSpec((C, C), lambda i: (0, 0)),
            pl.BlockSpec((RB,), lambda i: (i,)),
            pl.BlockSpec((RB,), lambda i: (i,)),
        ],
        out_specs=[
            pl.BlockSpec((RB, C), lambda i: (i, 0)),
            pl.BlockSpec((RB,), lambda i: (i,)),
        ],
        out_shape=[
            jax.ShapeDtypeStruct((N, C), jnp.float32),
            jax.ShapeDtypeStruct((NP,), jnp.float32),
        ],
    )(x, conv_w, deg0, deg1)


def _sc_agg(xws, src3, dst3, ew3, z2d):
    @functools.partial(
        pl.kernel,
        out_type=jax.ShapeDtypeStruct((NC, NP, C), jnp.float32),
        mesh=_sc_mesh(),
        compiler_params=_SC_PARAMS,
        scratch_types=[
            pltpu.VMEM((9 * B,), jnp.int32),
            pltpu.VMEM((9 * B,), jnp.int32),
            pltpu.VMEM((9 * B,), jnp.float32),
            pltpu.VMEM((3, B), jnp.int32),
            pltpu.VMEM((B,), jnp.float32),
            pltpu.VMEM((3, B, C), jnp.float32),
            pltpu.VMEM_SHARED((NP, C), jnp.float32),
            pltpu.SemaphoreType.DMA,
            pltpu.SemaphoreType.DMA,
            pltpu.SemaphoreType.DMA,
            pltpu.SemaphoreType.DMA,
            pltpu.SemaphoreType.DMA,
            pltpu.SemaphoreType.DMA,
            pltpu.SemaphoreType.DMA,
            pltpu.SemaphoreType.DMA,
            pltpu.SemaphoreType.DMA,
        ],
    )
    def k(xw_hbm, src_hbm, dst_hbm, ew_hbm, z_hbm, out_hbm,
          sidx_v, didx_v, ew_v, dbuf_v, ewb_v, gbuf, acc_sh,
          ps0, ps1, ps2, gs0, gs1, gs2, ss0, ss1, ss2):
        psems = (ps0, ps1, ps2)
        gsems = (gs0, gs1, gs2)
        ssems = (ss0, ss1, ss2)
        cid = lax.axis_index("c")
        sid = lax.axis_index("s")
        wid = sid * NC + cid
        r0 = sid * RPT
        SBW = 3 * B
        ebase = wid * EPW

        def idx_fetch(hoff, base, copy):
            copy(src_hbm.at[pl.ds(ebase + hoff, SBW)],
                 sidx_v.at[pl.ds(base, SBW)])
            copy(dst_hbm.at[pl.ds(ebase + hoff, SBW)],
                 didx_v.at[pl.ds(base, SBW)])
            copy(ew_hbm.at[pl.ds(ebase + hoff, SBW)],
                 ew_v.at[pl.ds(base, SBW)])

        def psem_drain(sem):
            for r in (sidx_v, didx_v, ew_v):
                pltpu.make_async_copy(
                    src_hbm.at[pl.ds(0, SBW)], r.at[pl.ds(0, SBW)], sem
                ).wait()

        idx_fetch(0, 0, pltpu.sync_copy)
        idx_fetch(SBW, SBW, lambda s_, d_: pltpu.async_copy(s_, d_, psems[1]))
        idx_fetch(2 * SBW, 2 * SBW,
                  lambda s_, d_: pltpu.async_copy(s_, d_, psems[2]))
        pltpu.sync_copy(z_hbm.at[pl.ds(r0, RPT)], acc_sh.at[pl.ds(r0, RPT)])
        for i in range(2):
            pltpu.async_copy(
                xw_hbm.at[sidx_v.at[pl.ds(i * B, B)]], gbuf.at[i], gsems[i])
        plsc.subcore_barrier()

        def step(s, carry):
            par = lax.rem(s, 3) * SBW
            npar = lax.rem(s + 1, 3) * SBW

            for i in range(3):
                b = s * 3 + i
                boff = par + i * B
                for g in range(B // VL):
                    sl = pl.ds(boff + g * VL, VL)
                    csl = pl.ds(g * VL, VL)
                    ewb_v[csl] = ew_v[sl]
                    dbuf_v[i, csl] = didx_v[sl]
                pltpu.make_async_copy(
                    z_hbm.at[pl.ds(0, B)], gbuf.at[i], gsems[i]
                ).wait()
                for j in range(B):
                    spl = plsc.load_gather(
                        ewb_v, [jnp.full((VL,), j, jnp.int32)]
                    )
                    for cc in range(C // VL):
                        csl = pl.ds(cc * VL, VL)
                        gbuf[i, j, csl] = gbuf[i, j, csl] * spl
                pltpu.async_copy(
                    gbuf.at[i], acc_sh.at[dbuf_v.at[i]], ssems[i],
                    add=True
                )
                @pl.when(b >= 1)
                def _():
                    pltpu.make_async_copy(
                        z_hbm.at[pl.ds(0, B)], gbuf.at[(i + 2) % 3],
                        ssems[(i + 2) % 3]
                    ).wait()

                if i == 0:
                    rem_s = lax.rem(s, 3)
                    for rr in range(3):
                        @pl.when((rem_s == rr) & (s >= 1) & (s + 2 < SB))
                        def _(rr=rr):
                            idx_fetch(
                                (s + 2) * SBW, ((rr + 2) % 3) * SBW,
                                lambda s_, d_: pltpu.async_copy(
                                    s_, d_, psems[(rr + 2) % 3]))

                if i == 1:
                    rem_s = lax.rem(s, 3)
                    for rr in range(3):
                        @pl.when((rem_s == rr) & (s + 1 < SB))
                        def _(rr=rr):
                            psem_drain(psems[(rr + 1) % 3])

                @pl.when(b + 2 < NBLK)
                def _():
                    if i == 0:
                        noff = par + 2 * B
                    else:
                        noff = npar + (i - 1) * B
                    pltpu.async_copy(
                        xw_hbm.at[sidx_v.at[pl.ds(noff, B)]],
                        gbuf.at[(i + 2) % 3], gsems[(i + 2) % 3])
            return carry

        lax.fori_loop(0, SB, step, 0)
        pltpu.make_async_copy(
            z_hbm.at[pl.ds(0, B)], gbuf.at[(NBLK - 1) % 3],
            ssems[(NBLK - 1) % 3]
        ).wait()
        plsc.subcore_barrier()
        pltpu.sync_copy(
            acc_sh.at[pl.ds(r0, RPT)], out_hbm.at[cid, pl.ds(r0, RPT)]
        )

    return k(xws, src3, dst3, ew3, z2d)


def _tc_final(aggp, dis, h, conv_b, lin1_w, lin1_b, lin2_w, lin2_b):
    RB = 1024
    grid = NP // RB

    def body(a_ref, d_ref, h_ref, cb_ref, w1_ref, b1_ref, w2_ref, b2_ref,
             out_ref):
        f = jax.nn.sigmoid(
            d_ref[...][:, None] * (a_ref[0] + a_ref[1]) + cb_ref[...])
        hh = h_ref[...]
        cat1 = jnp.concatenate([f, hh], axis=1)
        ru = jax.nn.sigmoid(
            jnp.dot(cat1, w1_ref[...], preferred_element_type=jnp.float32)
            + b1_ref[...]
        )
        r = ru[:, :C]
        u = ru[:, C:]
        cat2 = jnp.concatenate([f, r * hh], axis=1)
        cnew = jnp.tanh(
            jnp.dot(cat2, w2_ref[...], preferred_element_type=jnp.float32)
            + b2_ref[...]
        )
        out_ref[...] = u * hh + (1.0 - u) * cnew

    return pl.pallas_call(
        body,
        grid=(grid,),
        in_specs=[
            pl.BlockSpec((NC, RB, C), lambda i: (0, i, 0)),
            pl.BlockSpec((RB,), lambda i: (i,)),
            pl.BlockSpec((RB, C), lambda i: (i, 0)),
            pl.BlockSpec((C,), lambda i: (0,)),
            pl.BlockSpec((2 * C, 2 * C), lambda i: (0, 0)),
            pl.BlockSpec((2 * C,), lambda i: (0,)),
            pl.BlockSpec((2 * C, C), lambda i: (0, 0)),
            pl.BlockSpec((C,), lambda i: (0,)),
        ],
        out_specs=pl.BlockSpec((RB, C), lambda i: (i, 0)),
        out_shape=jax.ShapeDtypeStruct((N, C), jnp.float32),
    )(aggp, dis, h, conv_b, lin1_w, lin1_b, lin2_w, lin2_b)


def kernel(x, edge_index, edge_weight, h, conv_w, conv_b,
           lin1_w, lin1_b, lin2_w, lin2_b):
    src = edge_index[0].astype(jnp.int32)
    dst = edge_index[1].astype(jnp.int32)
    loop = jnp.arange(N, dtype=jnp.int32)
    npad = EP - EF
    zi = jnp.zeros((npad,), jnp.int32)
    srcf = jnp.concatenate([src, loop, zi])
    dstf = jnp.concatenate([dst, loop, zi])
    ewf = jnp.concatenate([
        edge_weight.astype(jnp.float32),
        jnp.full((N,), 2.0, jnp.float32),
        jnp.zeros((npad,), jnp.float32),
    ])
    dst3 = dstf.reshape(NW, NBLK, B)
    ew3 = ewf.reshape(NW, NBLK, B)
    z1d = jnp.zeros((NP,), jnp.float32)
    z2d = jnp.zeros((NP, C), jnp.float32)

    deg0, deg1 = _sc_deg(dst3, ew3, z1d)
    xws, dis = _tc_prep(x, conv_w, deg0, deg1)
    aggp = _sc_agg(xws, srcf, dstf, ewf, z2d)
    return _tc_final(aggp, dis, h, conv_b, lin1_w, lin1_b, lin2_w, lin2_b)


# speedup vs baseline: 1.5410x; 1.5410x over previous
import functools

import jax
import jax.numpy as jnp
from jax import lax
from jax.experimental import pallas as pl
from jax.experimental.pallas import tpu as pltpu
from jax.experimental.pallas import tpu_sc as plsc

N = 10000
C = 128
E = 320000
NC = 2
NS = 16
NW = NC * NS
NP = 10240
RPT = NP // NS
EF = E + N
EP = 330240
EPW = EP // NW
B = 80
NBLK = EPW // B
SB = NBLK // 3
VL = 16


def _sc_mesh():
    return plsc.VectorSubcoreMesh(
        core_axis_name="c", subcore_axis_name="s", num_cores=NC, num_subcores=NS
    )


_SC_PARAMS = pltpu.CompilerParams(needs_layout_passes=False)


def _sc_deg(dst3, ew3, z1d):
    @functools.partial(
        pl.kernel,
        out_type=(
            jax.ShapeDtypeStruct((NP,), jnp.float32),
            jax.ShapeDtypeStruct((NP,), jnp.float32),
        ),
        mesh=_sc_mesh(),
        compiler_params=_SC_PARAMS,
        scratch_types=[
            pltpu.VMEM((NBLK, B), jnp.int32),
            pltpu.VMEM((NBLK, B), jnp.float32),
            pltpu.VMEM_SHARED((NP,), jnp.float32),
            pltpu.SemaphoreType.DMA,
        ],
    )
    def k(dst_hbm, ew_hbm, z_hbm, out0, out1, didx_v, ew_v, deg_sh, ssem):
        cid = lax.axis_index("c")
        sid = lax.axis_index("s")
        wid = sid * NC + cid
        r0 = sid * RPT
        pltpu.sync_copy(dst_hbm.at[wid], didx_v)
        pltpu.sync_copy(ew_hbm.at[wid], ew_v)
        pltpu.sync_copy(z_hbm.at[pl.ds(r0, RPT)], deg_sh.at[pl.ds(r0, RPT)])
        plsc.subcore_barrier()

        def body(b, carry):
            pltpu.async_copy(
                ew_v.at[b], deg_sh.at[didx_v.at[b]], ssem, add=True
            )
            @pl.when(b >= 8)
            def _():
                pltpu.make_async_copy(
                    z_hbm.at[pl.ds(0, B)], ew_v.at[0], ssem
                ).wait()
            return carry

        lax.fori_loop(0, NBLK, body, 0)
        for _ in range(8):
            pltpu.make_async_copy(
                z_hbm.at[pl.ds(0, B)], ew_v.at[0], ssem
            ).wait()
        plsc.subcore_barrier()

        @pl.when(cid == 0)
        def _():
            pltpu.sync_copy(deg_sh.at[pl.ds(r0, RPT)], out0.at[pl.ds(r0, RPT)])

        @pl.when(cid == 1)
        def _():
            pltpu.sync_copy(deg_sh.at[pl.ds(r0, RPT)], out1.at[pl.ds(r0, RPT)])

    return k(dst3, ew3, z1d)


def _tc_prep(x, conv_w, deg0, deg1):
    RB = 1000
    grid = N // RB
    DB = NP // grid

    def body(x_ref, w_ref, d0_ref, d1_ref, xw_ref, dis_ref):
        xw_ref[...] = jnp.dot(
            x_ref[...], w_ref[...], preferred_element_type=jnp.float32
        )
        deg = d0_ref[...] + d1_ref[...]
        dis_ref[...] = lax.rsqrt(jnp.maximum(deg, 1e-12))

    return pl.pallas_call(
        body,
        grid=(grid,),
        in_specs=[
            pl.BlockSpec((RB, C), lambda i: (i, 0)),
            pl.BlockSpec((C, C), lambda i: (0, 0)),
            pl.BlockSpec((DB,), lambda i: (i,)),
            pl.BlockSpec((DB,), lambda i: (i,)),
        ],
        out_specs=[
            pl.BlockSpec((RB, C), lambda i: (i, 0)),
            pl.BlockSpec((DB,), lambda i: (i,)),
        ],
        out_shape=[
            jax.ShapeDtypeStruct((N, C), jnp.float32),
            jax.ShapeDtypeStruct((NP,), jnp.float32),
        ],
    )(x, conv_w, deg0, deg1)


def _sc_agg(xw, srcf, dstf, ewf, dis, z2d):
    @functools.partial(
        pl.kernel,
        out_type=jax.ShapeDtypeStruct((NC, NP, C), jnp.float32),
        mesh=_sc_mesh(),
        compiler_params=_SC_PARAMS,
        scratch_types=[
            pltpu.VMEM((NP,), jnp.float32),
            pltpu.VMEM((9 * B,), jnp.int32),
            pltpu.VMEM((9 * B,), jnp.int32),
            pltpu.VMEM((9 * B,), jnp.float32),
            pltpu.VMEM((3, B), jnp.int32),
            pltpu.VMEM((B,), jnp.float32),
            pltpu.VMEM((3, B, C), jnp.float32),
            pltpu.VMEM_SHARED((NP, C), jnp.float32),
            pltpu.SemaphoreType.DMA,
            pltpu.SemaphoreType.DMA,
            pltpu.SemaphoreType.DMA,
            pltpu.SemaphoreType.DMA,
            pltpu.SemaphoreType.DMA,
            pltpu.SemaphoreType.DMA,
            pltpu.SemaphoreType.DMA,
            pltpu.SemaphoreType.DMA,
            pltpu.SemaphoreType.DMA,
        ],
    )
    def k(xw_hbm, src_hbm, dst_hbm, ew_hbm, dis_hbm, z_hbm, out_hbm,
          dis_v, sidx_v, didx_v, ew_v, dbuf_v, norm_v, gbuf, acc_sh,
          ps0, ps1, ps2, gs0, gs1, gs2, ss0, ss1, ss2):
        psems = (ps0, ps1, ps2)
        gsems = (gs0, gs1, gs2)
        ssems = (ss0, ss1, ss2)
        cid = lax.axis_index("c")
        sid = lax.axis_index("s")
        wid = sid * NC + cid
        r0 = sid * RPT
        SBW = 3 * B
        ebase = wid * EPW

        def idx_fetch(hoff, base, copy):
            copy(src_hbm.at[pl.ds(ebase + hoff, SBW)],
                 sidx_v.at[pl.ds(base, SBW)])
            copy(dst_hbm.at[pl.ds(ebase + hoff, SBW)],
                 didx_v.at[pl.ds(base, SBW)])
            copy(ew_hbm.at[pl.ds(ebase + hoff, SBW)],
                 ew_v.at[pl.ds(base, SBW)])

        def psem_drain(sem):
            for r in (sidx_v, didx_v, ew_v):
                pltpu.make_async_copy(
                    src_hbm.at[pl.ds(0, SBW)], r.at[pl.ds(0, SBW)], sem
                ).wait()

        pltpu.sync_copy(dis_hbm, dis_v)
        idx_fetch(0, 0, pltpu.sync_copy)
        idx_fetch(SBW, SBW, lambda s_, d_: pltpu.async_copy(s_, d_, psems[1]))
        idx_fetch(2 * SBW, 2 * SBW,
                  lambda s_, d_: pltpu.async_copy(s_, d_, psems[2]))
        pltpu.sync_copy(z_hbm.at[pl.ds(r0, RPT)], acc_sh.at[pl.ds(r0, RPT)])
        for i in range(2):
            pltpu.async_copy(
                xw_hbm.at[sidx_v.at[pl.ds(i * B, B)]], gbuf.at[i], gsems[i])
        plsc.subcore_barrier()

        def step(s, carry):
            par = lax.rem(s, 3) * SBW
            npar = lax.rem(s + 1, 3) * SBW

            for i in range(3):
                b = s * 3 + i
                boff = par + i * B
                for g in range(B // VL):
                    sl = pl.ds(boff + g * VL, VL)
                    n16 = (
                        plsc.load_gather(dis_v, [sidx_v[sl]])
                        * ew_v[sl]
                        * plsc.load_gather(dis_v, [didx_v[sl]])
                    )
                    norm_v[pl.ds(g * VL, VL)] = n16
                pltpu.make_async_copy(
                    z_hbm.at[pl.ds(0, B)], gbuf.at[i], gsems[i]
                ).wait()
                for g in range(B // VL):
                    dbuf_v[i, pl.ds(g * VL, VL)] = (
                        didx_v[pl.ds(boff + g * VL, VL)])
                for j in range(B):
                    spl = plsc.load_gather(
                        norm_v, [jnp.full((VL,), j, jnp.int32)]
                    )
                    for cc in range(C // VL):
                        csl = pl.ds(cc * VL, VL)
                        gbuf[i, j, csl] = gbuf[i, j, csl] * spl
                @pl.when(b >= 1)
                def _():
                    pltpu.make_async_copy(
                        z_hbm.at[pl.ds(0, B)], gbuf.at[(i + 2) % 3],
                        ssems[(i + 2) % 3]
                    ).wait()

                if i == 0:
                    rem_s = lax.rem(s, 3)
                    for rr in range(3):
                        @pl.when((rem_s == rr) & (s >= 1) & (s + 2 < SB))
                        def _(rr=rr):
                            idx_fetch(
                                (s + 2) * SBW, ((rr + 2) % 3) * SBW,
                                lambda s_, d_: pltpu.async_copy(
                                    s_, d_, psems[(rr + 2) % 3]))

                if i == 1:
                    rem_s = lax.rem(s, 3)
                    for rr in range(3):
                        @pl.when((rem_s == rr) & (s + 1 < SB))
                        def _(rr=rr):
                            psem_drain(psems[(rr + 1) % 3])

                @pl.when(b + 2 < NBLK)
                def _():
                    if i == 0:
                        noff = par + 2 * B
                    else:
                        noff = npar + (i - 1) * B
                    pltpu.async_copy(
                        xw_hbm.at[sidx_v.at[pl.ds(noff, B)]],
                        gbuf.at[(i + 2) % 3], gsems[(i + 2) % 3])
                pltpu.async_copy(
                    gbuf.at[i], acc_sh.at[dbuf_v.at[i]], ssems[i],
                    add=True
                )
            return carry

        lax.fori_loop(0, SB, step, 0)
        pltpu.make_async_copy(
            z_hbm.at[pl.ds(0, B)], gbuf.at[(NBLK - 1) % 3],
            ssems[(NBLK - 1) % 3]
        ).wait()
        plsc.subcore_barrier()
        pltpu.sync_copy(
            acc_sh.at[pl.ds(r0, RPT)], out_hbm.at[cid, pl.ds(r0, RPT)]
        )

    return k(xw, srcf, dstf, ewf, dis, z2d)


def _tc_final(aggp, h, conv_b, lin1_w, lin1_b, lin2_w, lin2_b):
    RB = 1000
    grid = N // RB

    def body(a_ref, h_ref, cb_ref, w1_ref, b1_ref, w2_ref, b2_ref, out_ref):
        f = jax.nn.sigmoid(a_ref[0] + a_ref[1] + cb_ref[...])
        hh = h_ref[...]
        cat1 = jnp.concatenate([f, hh], axis=1)
        ru = jax.nn.sigmoid(
            jnp.dot(cat1, w1_ref[...], preferred_element_type=jnp.float32)
            + b1_ref[...]
        )
        r = ru[:, :C]
        u = ru[:, C:]
        cat2 = jnp.concatenate([f, r * hh], axis=1)
        cnew = jnp.tanh(
            jnp.dot(cat2, w2_ref[...], preferred_element_type=jnp.float32)
            + b2_ref[...]
        )
        out_ref[...] = u * hh + (1.0 - u) * cnew

    return pl.pallas_call(
        body,
        grid=(grid,),
        in_specs=[
            pl.BlockSpec((NC, RB, C), lambda i: (0, i, 0)),
            pl.BlockSpec((RB, C), lambda i: (i, 0)),
            pl.BlockSpec((C,), lambda i: (0,)),
            pl.BlockSpec((2 * C, 2 * C), lambda i: (0, 0)),
            pl.BlockSpec((2 * C,), lambda i: (0,)),
            pl.BlockSpec((2 * C, C), lambda i: (0, 0)),
            pl.BlockSpec((C,), lambda i: (0,)),
        ],
        out_specs=pl.BlockSpec((RB, C), lambda i: (i, 0)),
        out_shape=jax.ShapeDtypeStruct((N, C), jnp.float32),
    )(aggp, h, conv_b, lin1_w, lin1_b, lin2_w, lin2_b)


def kernel(x, edge_index, edge_weight, h, conv_w, conv_b,
           lin1_w, lin1_b, lin2_w, lin2_b):
    src = edge_index[0].astype(jnp.int32)
    dst = edge_index[1].astype(jnp.int32)
    loop = jnp.arange(N, dtype=jnp.int32)
    npad = EP - EF
    zi = jnp.zeros((npad,), jnp.int32)
    srcf = jnp.concatenate([src, loop, zi])
    dstf = jnp.concatenate([dst, loop, zi])
    ewf = jnp.concatenate([
        edge_weight.astype(jnp.float32),
        jnp.full((N,), 2.0, jnp.float32),
        jnp.zeros((npad,), jnp.float32),
    ])
    dst3 = dstf.reshape(NW, NBLK, B)
    ew3 = ewf.reshape(NW, NBLK, B)
    z1d = jnp.zeros((NP,), jnp.float32)
    z2d = jnp.zeros((NP, C), jnp.float32)

    deg0, deg1 = _sc_deg(dst3, ew3, z1d)
    xw, dis = _tc_prep(x, conv_w, deg0, deg1)
    aggp = _sc_agg(xw, srcf, dstf, ewf, dis, z2d)
    return _tc_final(aggp, h, conv_b, lin1_w, lin1_b, lin2_w, lin2_b)


# speedup vs baseline: 2.6965x; 1.7498x over previous
import functools

import jax
import jax.numpy as jnp
from jax import lax
from jax.experimental import pallas as pl
from jax.experimental.pallas import tpu as pltpu
from jax.experimental.pallas import tpu_sc as plsc

N = 10000
C = 128
E = 320000
NC = 2
NS = 16
NW = NC * NS
NP = 10240
RPT = NP // NS
EF = E + N
EP = 330240
EPW = EP // NW
B = 80
NBLK = EPW // B
SB = NBLK // 3
VL = 16


def _sc_mesh():
    return plsc.VectorSubcoreMesh(
        core_axis_name="c", subcore_axis_name="s", num_cores=NC, num_subcores=NS
    )


_SC_PARAMS = pltpu.CompilerParams(needs_layout_passes=False)


def _sc_deg(dst3, ew3, z1d):
    @functools.partial(
        pl.kernel,
        out_type=(
            jax.ShapeDtypeStruct((NP,), jnp.float32),
            jax.ShapeDtypeStruct((NP,), jnp.float32),
        ),
        mesh=_sc_mesh(),
        compiler_params=_SC_PARAMS,
        scratch_types=[
            pltpu.VMEM((NBLK, B), jnp.int32),
            pltpu.VMEM((NBLK, B), jnp.float32),
            pltpu.VMEM_SHARED((NP,), jnp.float32),
            pltpu.SemaphoreType.DMA,
        ],
    )
    def k(dst_hbm, ew_hbm, z_hbm, out0, out1, didx_v, ew_v, deg_sh, ssem):
        cid = lax.axis_index("c")
        sid = lax.axis_index("s")
        wid = sid * NC + cid
        r0 = sid * RPT
        pltpu.sync_copy(dst_hbm.at[wid], didx_v)
        pltpu.sync_copy(ew_hbm.at[wid], ew_v)
        pltpu.sync_copy(z_hbm.at[pl.ds(r0, RPT)], deg_sh.at[pl.ds(r0, RPT)])
        plsc.subcore_barrier()

        def body(b, carry):
            pltpu.async_copy(
                ew_v.at[b], deg_sh.at[didx_v.at[b]], ssem, add=True
            )
            @pl.when(b >= 8)
            def _():
                pltpu.make_async_copy(
                    z_hbm.at[pl.ds(0, B)], ew_v.at[0], ssem
                ).wait()
            return carry

        lax.fori_loop(0, NBLK, body, 0)
        for _ in range(8):
            pltpu.make_async_copy(
                z_hbm.at[pl.ds(0, B)], ew_v.at[0], ssem
            ).wait()
        plsc.subcore_barrier()

        @pl.when(cid == 0)
        def _():
            pltpu.sync_copy(deg_sh.at[pl.ds(r0, RPT)], out0.at[pl.ds(r0, RPT)])

        @pl.when(cid == 1)
        def _():
            pltpu.sync_copy(deg_sh.at[pl.ds(r0, RPT)], out1.at[pl.ds(r0, RPT)])

    return k(dst3, ew3, z1d)


def _tc_prep(x, conv_w, deg0, deg1):
    RB = 1000
    grid = N // RB
    DB = NP // grid

    def body(x_ref, w_ref, d0_ref, d1_ref, xw_ref, dis_ref):
        xw_ref[...] = jnp.dot(
            x_ref[...], w_ref[...], preferred_element_type=jnp.float32
        )
        deg = d0_ref[...] + d1_ref[...]
        dis_ref[...] = lax.rsqrt(jnp.maximum(deg, 1e-12))

    return pl.pallas_call(
        body,
        grid=(grid,),
        in_specs=[
            pl.BlockSpec((RB, C), lambda i: (i, 0)),
            pl.BlockSpec((C, C), lambda i: (0, 0)),
            pl.BlockSpec((DB,), lambda i: (i,)),
            pl.BlockSpec((DB,), lambda i: (i,)),
        ],
        out_specs=[
            pl.BlockSpec((RB, C), lambda i: (i, 0)),
            pl.BlockSpec((DB,), lambda i: (i,)),
        ],
        out_shape=[
            jax.ShapeDtypeStruct((N, C), jnp.float32),
            jax.ShapeDtypeStruct((NP,), jnp.float32),
        ],
    )(x, conv_w, deg0, deg1)


def _sc_agg(xw, srcf, dstf, ewf, dis, z2d):
    @functools.partial(
        pl.kernel,
        out_type=jax.ShapeDtypeStruct((NC, NP, C), jnp.float32),
        mesh=_sc_mesh(),
        compiler_params=_SC_PARAMS,
        scratch_types=[
            pltpu.VMEM((NP,), jnp.float32),
            pltpu.VMEM((9 * B,), jnp.int32),
            pltpu.VMEM((9 * B,), jnp.int32),
            pltpu.VMEM((9 * B,), jnp.float32),
            pltpu.VMEM((3, B), jnp.int32),
            pltpu.VMEM((B,), jnp.float32),
            pltpu.VMEM((3, B, C), jnp.float32),
            pltpu.VMEM_SHARED((NP, C), jnp.float32),
            pltpu.SemaphoreType.DMA,
            pltpu.SemaphoreType.DMA,
            pltpu.SemaphoreType.DMA,
            pltpu.SemaphoreType.DMA,
            pltpu.SemaphoreType.DMA,
            pltpu.SemaphoreType.DMA,
            pltpu.SemaphoreType.DMA,
            pltpu.SemaphoreType.DMA,
            pltpu.SemaphoreType.DMA,
        ],
    )
    def k(xw_hbm, src_hbm, dst_hbm, ew_hbm, dis_hbm, z_hbm, out_hbm,
          dis_v, sidx_v, didx_v, ew_v, dbuf_v, norm_v, gbuf, acc_sh,
          ps0, ps1, ps2, gs0, gs1, gs2, ss0, ss1, ss2):
        psems = (ps0, ps1, ps2)
        gsems = (gs0, gs1, gs2)
        ssems = (ss0, ss1, ss2)
        cid = lax.axis_index("c")
        sid = lax.axis_index("s")
        wid = sid * NC + cid
        r0 = sid * RPT
        SBW = 3 * B
        ebase = wid * EPW

        def idx_fetch(hoff, base, copy):
            copy(src_hbm.at[pl.ds(ebase + hoff, SBW)],
                 sidx_v.at[pl.ds(base, SBW)])
            copy(dst_hbm.at[pl.ds(ebase + hoff, SBW)],
                 didx_v.at[pl.ds(base, SBW)])
            copy(ew_hbm.at[pl.ds(ebase + hoff, SBW)],
                 ew_v.at[pl.ds(base, SBW)])

        def psem_drain(sem):
            for r in (sidx_v, didx_v, ew_v):
                pltpu.make_async_copy(
                    src_hbm.at[pl.ds(0, SBW)], r.at[pl.ds(0, SBW)], sem
                ).wait()

        pltpu.sync_copy(dis_hbm, dis_v)
        idx_fetch(0, 0, pltpu.sync_copy)
        idx_fetch(SBW, SBW, lambda s_, d_: pltpu.async_copy(s_, d_, psems[1]))
        idx_fetch(2 * SBW, 2 * SBW,
                  lambda s_, d_: pltpu.async_copy(s_, d_, psems[2]))
        pltpu.sync_copy(z_hbm.at[pl.ds(r0, RPT)], acc_sh.at[pl.ds(r0, RPT)])
        for i in range(2):
            pltpu.async_copy(
                xw_hbm.at[sidx_v.at[pl.ds(i * B, B)]], gbuf.at[i], gsems[i])
        plsc.subcore_barrier()

        def step(s, carry):
            par = lax.rem(s, 3) * SBW
            npar = lax.rem(s + 1, 3) * SBW

            for i in range(3):
                b = s * 3 + i
                boff = par + i * B
                for g in range(B // VL):
                    sl = pl.ds(boff + g * VL, VL)
                    n16 = (
                        plsc.load_gather(dis_v, [sidx_v[sl]])
                        * ew_v[sl]
                        * plsc.load_gather(dis_v, [didx_v[sl]])
                    )
                    norm_v[pl.ds(g * VL, VL)] = n16
                pltpu.make_async_copy(
                    z_hbm.at[pl.ds(0, B)], gbuf.at[i], gsems[i]
                ).wait()
                for g in range(B // VL):
                    dbuf_v[i, pl.ds(g * VL, VL)] = (
                        didx_v[pl.ds(boff + g * VL, VL)])
                @plsc.parallel_loop(0, B, step=1, unroll=8)
                def _(j):
                    spl = plsc.load_gather(
                        norm_v, [jnp.full((VL,), j, jnp.int32)]
                    )
                    for cc in range(C // VL):
                        csl = pl.ds(cc * VL, VL)
                        gbuf[i, j, csl] = gbuf[i, j, csl] * spl
                @pl.when(b >= 1)
                def _():
                    pltpu.make_async_copy(
                        z_hbm.at[pl.ds(0, B)], gbuf.at[(i + 2) % 3],
                        ssems[(i + 2) % 3]
                    ).wait()

                if i == 0:
                    rem_s = lax.rem(s, 3)
                    for rr in range(3):
                        @pl.when((rem_s == rr) & (s >= 1) & (s + 2 < SB))
                        def _(rr=rr):
                            idx_fetch(
                                (s + 2) * SBW, ((rr + 2) % 3) * SBW,
                                lambda s_, d_: pltpu.async_copy(
                                    s_, d_, psems[(rr + 2) % 3]))

                if i == 1:
                    rem_s = lax.rem(s, 3)
                    for rr in range(3):
                        @pl.when((rem_s == rr) & (s + 1 < SB))
                        def _(rr=rr):
                            psem_drain(psems[(rr + 1) % 3])

                @pl.when(b + 2 < NBLK)
                def _():
                    if i == 0:
                        noff = par + 2 * B
                    else:
                        noff = npar + (i - 1) * B
                    pltpu.async_copy(
                        xw_hbm.at[sidx_v.at[pl.ds(noff, B)]],
                        gbuf.at[(i + 2) % 3], gsems[(i + 2) % 3])
                pltpu.async_copy(
                    gbuf.at[i], acc_sh.at[dbuf_v.at[i]], ssems[i],
                    add=True
                )
            return carry

        lax.fori_loop(0, SB, step, 0)
        pltpu.make_async_copy(
            z_hbm.at[pl.ds(0, B)], gbuf.at[(NBLK - 1) % 3],
            ssems[(NBLK - 1) % 3]
        ).wait()
        plsc.subcore_barrier()
        pltpu.sync_copy(
            acc_sh.at[pl.ds(r0, RPT)], out_hbm.at[cid, pl.ds(r0, RPT)]
        )

    return k(xw, srcf, dstf, ewf, dis, z2d)


def _tc_final(aggp, h, conv_b, lin1_w, lin1_b, lin2_w, lin2_b):
    RB = 1000
    grid = N // RB

    def body(a_ref, h_ref, cb_ref, w1_ref, b1_ref, w2_ref, b2_ref, out_ref):
        f = jax.nn.sigmoid(a_ref[0] + a_ref[1] + cb_ref[...])
        hh = h_ref[...]
        cat1 = jnp.concatenate([f, hh], axis=1)
        ru = jax.nn.sigmoid(
            jnp.dot(cat1, w1_ref[...], preferred_element_type=jnp.float32)
            + b1_ref[...]
        )
        r = ru[:, :C]
        u = ru[:, C:]
        cat2 = jnp.concatenate([f, r * hh], axis=1)
        cnew = jnp.tanh(
            jnp.dot(cat2, w2_ref[...], preferred_element_type=jnp.float32)
            + b2_ref[...]
        )
        out_ref[...] = u * hh + (1.0 - u) * cnew

    return pl.pallas_call(
        body,
        grid=(grid,),
        in_specs=[
            pl.BlockSpec((NC, RB, C), lambda i: (0, i, 0)),
            pl.BlockSpec((RB, C), lambda i: (i, 0)),
            pl.BlockSpec((C,), lambda i: (0,)),
            pl.BlockSpec((2 * C, 2 * C), lambda i: (0, 0)),
            pl.BlockSpec((2 * C,), lambda i: (0,)),
            pl.BlockSpec((2 * C, C), lambda i: (0, 0)),
            pl.BlockSpec((C,), lambda i: (0,)),
        ],
        out_specs=pl.BlockSpec((RB, C), lambda i: (i, 0)),
        out_shape=jax.ShapeDtypeStruct((N, C), jnp.float32),
    )(aggp, h, conv_b, lin1_w, lin1_b, lin2_w, lin2_b)


def kernel(x, edge_index, edge_weight, h, conv_w, conv_b,
           lin1_w, lin1_b, lin2_w, lin2_b):
    src = edge_index[0].astype(jnp.int32)
    dst = edge_index[1].astype(jnp.int32)
    loop = jnp.arange(N, dtype=jnp.int32)
    npad = EP - EF
    zi = jnp.zeros((npad,), jnp.int32)
    srcf = jnp.concatenate([src, loop, zi])
    dstf = jnp.concatenate([dst, loop, zi])
    ewf = jnp.concatenate([
        edge_weight.astype(jnp.float32),
        jnp.full((N,), 2.0, jnp.float32),
        jnp.zeros((npad,), jnp.float32),
    ])
    dst3 = dstf.reshape(NW, NBLK, B)
    ew3 = ewf.reshape(NW, NBLK, B)
    z1d = jnp.zeros((NP,), jnp.float32)
    z2d = jnp.zeros((NP, C), jnp.float32)

    deg0, deg1 = _sc_deg(dst3, ew3, z1d)
    xw, dis = _tc_prep(x, conv_w, deg0, deg1)
    aggp = _sc_agg(xw, srcf, dstf, ewf, dis, z2d)
    return _tc_final(aggp, h, conv_b, lin1_w, lin1_b, lin2_w, lin2_b)
